# baseline (speedup 1.0000x reference)
import jax
import jax.numpy as jnp
from jax.experimental import pallas as pl
from jax.experimental.pallas import tpu as pltpu

_TI = 400


def _gc_body(x_ref, adj_ref, w_ref, out_ref, xt_ref):
    i = pl.program_id(0)

    @pl.when(i == 0)
    def _init():
        xt_ref[...] = x_ref[...].T
        out_ref[...] = jnp.zeros_like(out_ref)

    a_blk = jax.lax.dot_general(
        adj_ref[...], xt_ref[...],
        (((1,), (0,)), ((), ())),
        preferred_element_type=jnp.float32,
    )
    out_ref[...] += jax.lax.dot_general(
        a_blk, w_ref[...],
        (((0,), (0,)), ((), ())),
        preferred_element_type=jnp.float32,
    )


def kernel(x, adj, weight):
    d, n = x.shape
    f = weight.shape[1]
    grid = (n // _TI,)
    return pl.pallas_call(
        _gc_body,
        grid=grid,
        in_specs=[
            pl.BlockSpec((d, n), lambda i: (0, 0)),
            pl.BlockSpec((_TI, n), lambda i: (i, 0)),
            pl.BlockSpec((_TI, f), lambda i: (i, 0)),
        ],
        out_specs=pl.BlockSpec((d, f), lambda i: (0, 0)),
        out_shape=jax.ShapeDtypeStruct((d, f), jnp.float32),
        scratch_shapes=[pltpu.VMEM((n, d), jnp.float32)],
        compiler_params=pltpu.CompilerParams(
            dimension_semantics=("arbitrary",),
        ),
    )(x, adj, weight)


# speedup vs baseline: 1.0625x; 1.0625x over previous
import jax
import jax.numpy as jnp
from jax.experimental import pallas as pl
from jax.experimental.pallas import tpu as pltpu

_TI = 200
_R = 4


def _gc_body(xt_ref, adj_ref, w_ref, out_ref, buf_ref, sem_ref):
    i = pl.program_id(0)
    k = pl.num_programs(0)

    def copy(step, slot):
        return pltpu.make_async_copy(
            adj_ref.at[pl.ds(step * _TI, _TI), :],
            buf_ref.at[slot],
            sem_ref.at[slot],
        )

    @pl.when(i == 0)
    def _init():
        out_ref[...] = jnp.zeros_like(out_ref)
        for r in range(_R):
            copy(r, r).start()

    slot = jax.lax.rem(i, _R)
    copy(i, slot).wait()

    a_blk = jax.lax.dot_general(
        buf_ref[slot], xt_ref[...],
        (((1,), (0,)), ((), ())),
        preferred_element_type=jnp.float32,
    )
    out_ref[...] += jax.lax.dot_general(
        a_blk, w_ref[pl.ds(i * _TI, _TI), :],
        (((0,), (0,)), ((), ())),
        preferred_element_type=jnp.float32,
    )

    nxt = i + _R

    @pl.when(nxt < k)
    def _prefetch():
        copy(nxt, slot).start()


def kernel(x, adj, weight):
    d, n = x.shape
    f = weight.shape[1]
    xt = x.T
    grid = (n // _TI,)
    return pl.pallas_call(
        _gc_body,
        grid=grid,
        in_specs=[
            pl.BlockSpec((n, d), lambda i: (0, 0)),
            pl.BlockSpec(memory_space=pl.ANY),
            pl.BlockSpec((n, f), lambda i: (0, 0)),
        ],
        out_specs=pl.BlockSpec((d, f), lambda i: (0, 0)),
        out_shape=jax.ShapeDtypeStruct((d, f), jnp.float32),
        scratch_shapes=[
            pltpu.VMEM((_R, _TI, n), jnp.float32),
            pltpu.SemaphoreType.DMA((_R,)),
        ],
        compiler_params=pltpu.CompilerParams(
            dimension_semantics=("arbitrary",),
        ),
    )(xt, adj, weight)
